# baseline (speedup 1.0000x reference)
import functools

import jax
import jax.numpy as jnp
from jax import lax
from jax.experimental import pallas as pl
from jax.experimental.pallas import tpu as pltpu
from jax.experimental.pallas import tpu_sc as plsc

_NC = 2
_NS = 16
_LANES = 16
_DD = 16
_NBUF = 4


@functools.lru_cache(maxsize=None)
def _make_sc_aggregate(n_nodes, f, n_edges):
    fh = f // 2
    ew = n_edges // _NS
    cw = 1000
    nch = ew // cw
    assert ew * _NS == n_edges and nch * cw == ew and nch % 2 == 0
    bsizes = [128] * 7 + [104]
    boffs = [sum(bsizes[:i]) for i in range(len(bsizes))]
    nb = len(bsizes)
    assert sum(bsizes) == cw and nb % _NBUF == 0
    bmax = max(bsizes)
    drain_sz = [bsizes[(b - _NBUF) % nb] for b in range(nb)]
    nr = n_nodes // _NS
    assert n_nodes % _NS == 0
    zr = 25
    assert nr % zr == 0 and cw <= n_nodes
    mesh = plsc.VectorSubcoreMesh(core_axis_name="c", subcore_axis_name="s")

    @functools.partial(
        pl.kernel,
        mesh=mesh,
        compiler_params=pltpu.CompilerParams(use_tc_tiling_on_sc=False),
        out_type=[
            jax.ShapeDtypeStruct((n_nodes, 2 * f), jnp.float32),
            jax.ShapeDtypeStruct((_NC, n_nodes, f // 2), jnp.float32),
        ],
        scratch_types=[
            pltpu.VMEM((cw,), jnp.int32),
            pltpu.VMEM((cw,), jnp.int32),
            pltpu.VMEM((cw,), jnp.int32),
            pltpu.VMEM((cw,), jnp.int32),
            [pltpu.VMEM((bmax, fh), jnp.float32) for _ in range(_NBUF)],
            pltpu.VMEM((bmax, _DD), jnp.float32),
            pltpu.VMEM((zr, _DD), jnp.float32),
            pltpu.VMEM_SHARED((n_nodes, fh), jnp.float32),
            pltpu.VMEM_SHARED((n_nodes, _DD), jnp.float32),
            [pltpu.SemaphoreType.DMA for _ in range(_NBUF)],
            [pltpu.SemaphoreType.DMA for _ in range(_NBUF)],
            pltpu.SemaphoreType.DMA,
            pltpu.SemaphoreType.DMA,
            pltpu.SemaphoreType.DMA,
        ],
    )
    def sc_aggr(x_hbm, edge_hbm, outx_hbm, xh_hbm,
                src_a, dst_a, src_b, dst_b, rows, ones_v, zd,
                accx, accd, gs, ss, dsem, isa, isb):
        cid = lax.axis_index("c")
        sid = lax.axis_index("s")
        col0 = cid * fh

        def stage(c, sv, dv, sem):
            off = sid * ew + c * cw
            pltpu.async_copy(edge_hbm.at[0, pl.ds(off, cw)], sv, sem)
            pltpu.async_copy(edge_hbm.at[1, pl.ds(off, cw)], dv, sem)

        def wait_stage(sv, dv, sem):
            pltpu.make_async_copy(edge_hbm.at[0, pl.ds(0, cw)], sv,
                                  sem).wait()
            pltpu.make_async_copy(edge_hbm.at[0, pl.ds(0, cw)], dv,
                                  sem).wait()

        stage(0, src_a, dst_a, isa)
        stage(1, src_b, dst_b, isb)

        base = sid * nr
        nxc = nr // bmax + (1 if nr % bmax else 0)
        for q in range(nxc):
            r0 = q * bmax
            rn = min(bmax, nr - r0)
            pltpu.sync_copy(
                x_hbm.at[pl.ds(base + r0, rn), pl.ds(col0, fh)],
                rows[q % _NBUF].at[pl.ds(0, rn)])
            pltpu.sync_copy(
                rows[q % _NBUF].at[pl.ds(0, rn)],
                xh_hbm.at[cid, pl.ds(base + r0, rn)])

        def zrow(r, carry):
            def zcol(j, c2):
                for q in range(_NBUF):
                    rows[q][r, pl.ds(j * _LANES, _LANES)] = jnp.zeros(
                        (_LANES,), jnp.float32)
                return c2
            lax.fori_loop(0, fh // _LANES, zcol, carry)
            ones_v[r, pl.ds(0, _LANES)] = jnp.ones((_LANES,), jnp.float32)
            return carry
        lax.fori_loop(0, bmax, zrow, 0)

        def zdrow(r, carry):
            zd[r, pl.ds(0, _LANES)] = jnp.zeros((_LANES,), jnp.float32)
            return carry
        lax.fori_loop(0, zr, zdrow, 0)

        nzfull = nr // bmax
        for k in range(nzfull):
            pltpu.sync_copy(rows[k % _NBUF],
                            accx.at[pl.ds(base + k * bmax, bmax)])
        rem = nr - nzfull * bmax
        if rem:
            pltpu.sync_copy(rows[0].at[pl.ds(0, rem)],
                            accx.at[pl.ds(base + nzfull * bmax, rem)])
        for k in range(nr // zr):
            pltpu.sync_copy(zd, accd.at[pl.ds(base + k * zr, zr)])
        plsc.subcore_barrier()

        def process_chunk(sv, dv, mydeg, first, poststage=None):
            def process(k):
                jk = k % _NBUF
                sz = bsizes[k]
                pltpu.make_async_copy(
                    xh_hbm.at[0, pl.ds(0, sz)],
                    rows[jk].at[pl.ds(0, sz)], gs[jk]).wait()
                didx = dv.at[pl.ds(boffs[k], sz)]
                pltpu.async_copy(rows[jk].at[pl.ds(0, sz)],
                                 accx.at[didx], ss[jk], add=True)

                @pl.when(mydeg)
                def _():
                    pltpu.async_copy(ones_v.at[pl.ds(0, sz)],
                                     accd.at[didx], dsem, add=True)

            for b in range(nb):
                jb = b % _NBUF
                if not (first and b < _NBUF):
                    dsz = drain_sz[b]
                    pltpu.make_async_copy(
                        xh_hbm.at[0, pl.ds(0, dsz)],
                        rows[jb].at[pl.ds(0, dsz)], ss[jb]).wait()
                pltpu.async_copy(
                    xh_hbm.at[cid].at[sv.at[pl.ds(boffs[b], bsizes[b])]],
                    rows[jb].at[pl.ds(0, bsizes[b])], gs[jb])
                if b == _NBUF and poststage is not None:
                    poststage()
                if b >= 2:
                    process(b - 2)
            process(nb - 2)
            process(nb - 1)

            @pl.when(mydeg)
            def _():
                pltpu.make_async_copy(
                    xh_hbm.at[0, pl.ds(0, cw), pl.ds(0, _DD)],
                    accd.at[pl.ds(0, cw)], dsem).wait()

        def body_common(c0, first):
            def stage_b():
                @pl.when(c0 + 1 < nch)
                def _():
                    stage(c0 + 1, src_b, dst_b, isb)

            def stage_a():
                @pl.when(c0 + 2 < nch)
                def _():
                    stage(c0 + 2, src_a, dst_a, isa)

            wait_stage(src_a, dst_a, isa)
            process_chunk(src_a, dst_a, (c0 % 2) == cid, first,
                          None if first else stage_b)
            wait_stage(src_b, dst_b, isb)
            process_chunk(src_b, dst_b, ((c0 + 1) % 2) == cid, False,
                          stage_a)

        body_common(0, True)

        def body(cc, carry):
            body_common(2 * cc, False)
            return carry
        lax.fori_loop(1, nch // 2, body, 0)

        for j in range(_NBUF):
            sz = bsizes[nb - _NBUF + j]
            pltpu.make_async_copy(xh_hbm.at[0, pl.ds(0, sz)],
                                  rows[j].at[pl.ds(0, sz)], ss[j]).wait()
        plsc.subcore_barrier()

        pltpu.sync_copy(
            accx.at[pl.ds(base, nr)],
            outx_hbm.at[pl.ds(base, nr), pl.ds(col0, fh)])
        pltpu.sync_copy(
            accd.at[pl.ds(base, nr)],
            outx_hbm.at[pl.ds(base, nr), pl.ds(f + cid * _DD, _DD)])

    return sc_aggr


def _mlp(v, w1, b1, w2, b2):
    h = jnp.maximum(
        jnp.dot(v, w1[...], preferred_element_type=jnp.float32)
        + b1[...][None, :], 0.0)
    return (jnp.dot(h, w2[...], preferred_element_type=jnp.float32)
            + b2[...][None, :])


def _k_dense(n, f, *refs):
    (x_ref, ag_ref,
     c1w1, c1b1, c1w2, c1b2, c2w1, c2b1, c2w2, c2b2,
     c3w1, c3b1, c3w2, c3b2, mw1, mb1, mw2, mb2,
     u1w1, u1b1, u1w2, u1b2, u2w1, u2b1, u2w2, u2b2,
     u3w1, u3b1, u3w2, u3b2, out_ref) = refs
    y1 = x_ref[...] + ag_ref[:, 0:f]
    deg = (ag_ref[:, f:f + 1]
           + ag_ref[:, f + _DD:f + _DD + 1])
    x1 = _mlp(y1, c1w1, c1b1, c1w2, c1b2)
    xp1 = jnp.sum(x1, axis=0, keepdims=True) * (1.0 / n)
    m = 1.0 + deg[0:1, :]
    x2 = _mlp(m * xp1, c2w1, c2b1, c2w2, c2b2)
    x3 = _mlp(m * x2, c3w1, c3b1, c3w2, c3b2)
    xm = _mlp(m * x3, mw1, mb1, mw2, mb2)
    u1 = _mlp(m * jnp.concatenate([xm, x3], axis=1), u1w1, u1b1, u1w2, u1b2)
    u2 = _mlp(m * jnp.concatenate([u1, x2], axis=1), u2w1, u2b1, u2w2, u2b2)
    t1 = jnp.dot(u2, u3w1[0:64, :], preferred_element_type=jnp.float32)
    t0 = t1 + u3b1[...][None, :]
    z = jnp.dot(y1, u3w1[64:, :], preferred_element_type=jnp.float32)
    h = jnp.maximum(t0 + deg * t1 + z, 0.0)
    out_ref[...] = (jnp.dot(h, u3w2[...], preferred_element_type=jnp.float32)
                    + u3b2[...][None, :])


def kernel(x, edge_index, params):
    n, f = x.shape
    e = edge_index.shape[1]
    aggr, _ = _make_sc_aggregate(n, f, e)(x, edge_index)

    flat_w = []
    for name in ('c1', 'c2', 'c3', 'mid', 'u1', 'u2', 'u3'):
        flat_w += list(params[name])
    ins = [x, aggr] + flat_w
    specs = [pl.BlockSpec(a.shape,
                          functools.partial((lambda nd, j: (0,) * nd),
                                            a.ndim))
             for a in ins]
    out = pl.pallas_call(
        functools.partial(_k_dense, n, f),
        grid=(1,),
        in_specs=specs,
        out_specs=pl.BlockSpec((n, 64), lambda j: (0, 0)),
        out_shape=jax.ShapeDtypeStruct((n, 64), jnp.float32),
        compiler_params=pltpu.CompilerParams(
            vmem_limit_bytes=100 * 1024 * 1024),
    )(*ins)
    return out


# speedup vs baseline: 1.0353x; 1.0353x over previous
import functools

import jax
import jax.numpy as jnp
from jax import lax
from jax.experimental import pallas as pl
from jax.experimental.pallas import tpu as pltpu
from jax.experimental.pallas import tpu_sc as plsc

_NC = 2
_NS = 16
_LANES = 16
_DD = 16
_NBUF = 4


@functools.lru_cache(maxsize=None)
def _make_sc_aggregate(n_nodes, f, n_edges):
    fh = f // 2
    ew = n_edges // _NS
    cw = 1000
    nch = ew // cw
    assert ew * _NS == n_edges and nch * cw == ew and nch % 2 == 0
    bsizes = [128] * 7 + [104]
    boffs = [sum(bsizes[:i]) for i in range(len(bsizes))]
    nb = len(bsizes)
    assert sum(bsizes) == cw and nb % _NBUF == 0
    bmax = max(bsizes)
    drain_sz = [bsizes[(b - _NBUF) % nb] for b in range(nb)]
    nr = n_nodes // _NS
    assert n_nodes % _NS == 0
    zr = 25
    assert nr % zr == 0 and cw <= n_nodes
    mesh = plsc.VectorSubcoreMesh(core_axis_name="c", subcore_axis_name="s")

    @functools.partial(
        pl.kernel,
        mesh=mesh,
        compiler_params=pltpu.CompilerParams(use_tc_tiling_on_sc=False),
        out_type=[
            jax.ShapeDtypeStruct((n_nodes, f), jnp.float32),
            jax.ShapeDtypeStruct((_NC, n_nodes, _DD), jnp.float32),
            jax.ShapeDtypeStruct((_NC, n_nodes, f // 2), jnp.float32),
        ],
        scratch_types=[
            pltpu.VMEM((cw,), jnp.int32),
            pltpu.VMEM((cw,), jnp.int32),
            pltpu.VMEM((cw,), jnp.int32),
            pltpu.VMEM((cw,), jnp.int32),
            [pltpu.VMEM((bmax, fh), jnp.float32) for _ in range(_NBUF)],
            pltpu.VMEM((bmax, _DD), jnp.float32),
            pltpu.VMEM((zr, _DD), jnp.float32),
            pltpu.VMEM_SHARED((n_nodes, fh), jnp.float32),
            pltpu.VMEM_SHARED((n_nodes, _DD), jnp.float32),
            [pltpu.SemaphoreType.DMA for _ in range(_NBUF)],
            [pltpu.SemaphoreType.DMA for _ in range(_NBUF)],
            pltpu.SemaphoreType.DMA,
            pltpu.SemaphoreType.DMA,
            pltpu.SemaphoreType.DMA,
        ],
    )
    def sc_aggr(x_hbm, edge_hbm, outx_hbm, outd_hbm, xh_hbm,
                src_a, dst_a, src_b, dst_b, rows, ones_v, zd,
                accx, accd, gs, ss, dsem, isa, isb):
        cid = lax.axis_index("c")
        sid = lax.axis_index("s")
        col0 = cid * fh

        def stage(c, sv, dv, sem):
            off = sid * ew + c * cw
            pltpu.async_copy(edge_hbm.at[0, pl.ds(off, cw)], sv, sem)
            pltpu.async_copy(edge_hbm.at[1, pl.ds(off, cw)], dv, sem)

        def wait_stage(sv, dv, sem):
            pltpu.make_async_copy(edge_hbm.at[0, pl.ds(0, cw)], sv,
                                  sem).wait()
            pltpu.make_async_copy(edge_hbm.at[0, pl.ds(0, cw)], dv,
                                  sem).wait()

        stage(0, src_a, dst_a, isa)
        stage(1, src_b, dst_b, isb)

        base = sid * nr
        nxc = nr // bmax + (1 if nr % bmax else 0)
        for q in range(nxc):
            r0 = q * bmax
            rn = min(bmax, nr - r0)
            pltpu.sync_copy(
                x_hbm.at[pl.ds(base + r0, rn), pl.ds(col0, fh)],
                rows[q % _NBUF].at[pl.ds(0, rn)])
            pltpu.sync_copy(
                rows[q % _NBUF].at[pl.ds(0, rn)],
                xh_hbm.at[cid, pl.ds(base + r0, rn)])

        def zrow(r, carry):
            def zcol(j, c2):
                for q in range(_NBUF):
                    rows[q][r, pl.ds(j * _LANES, _LANES)] = jnp.zeros(
                        (_LANES,), jnp.float32)
                return c2
            lax.fori_loop(0, fh // _LANES, zcol, carry)
            ones_v[r, pl.ds(0, _LANES)] = jnp.ones((_LANES,), jnp.float32)
            return carry
        lax.fori_loop(0, bmax, zrow, 0)

        def zdrow(r, carry):
            zd[r, pl.ds(0, _LANES)] = jnp.zeros((_LANES,), jnp.float32)
            return carry
        lax.fori_loop(0, zr, zdrow, 0)

        nzfull = nr // bmax
        for k in range(nzfull):
            pltpu.sync_copy(rows[k % _NBUF],
                            accx.at[pl.ds(base + k * bmax, bmax)])
        rem = nr - nzfull * bmax
        if rem:
            pltpu.sync_copy(rows[0].at[pl.ds(0, rem)],
                            accx.at[pl.ds(base + nzfull * bmax, rem)])
        for k in range(nr // zr):
            pltpu.sync_copy(zd, accd.at[pl.ds(base + k * zr, zr)])
        plsc.subcore_barrier()

        def process_chunk(sv, dv, mydeg, first, poststage=None):
            def process(k):
                jk = k % _NBUF
                sz = bsizes[k]
                pltpu.make_async_copy(
                    xh_hbm.at[0, pl.ds(0, sz)],
                    rows[jk].at[pl.ds(0, sz)], gs[jk]).wait()
                didx = dv.at[pl.ds(boffs[k], sz)]
                pltpu.async_copy(rows[jk].at[pl.ds(0, sz)],
                                 accx.at[didx], ss[jk], add=True)

                @pl.when(mydeg)
                def _():
                    pltpu.async_copy(ones_v.at[pl.ds(0, sz)],
                                     accd.at[didx], dsem, add=True)

            for b in range(nb):
                jb = b % _NBUF
                if not (first and b < _NBUF):
                    dsz = drain_sz[b]
                    pltpu.make_async_copy(
                        xh_hbm.at[0, pl.ds(0, dsz)],
                        rows[jb].at[pl.ds(0, dsz)], ss[jb]).wait()
                pltpu.async_copy(
                    xh_hbm.at[cid].at[sv.at[pl.ds(boffs[b], bsizes[b])]],
                    rows[jb].at[pl.ds(0, bsizes[b])], gs[jb])
                if b == _NBUF and poststage is not None:
                    poststage()
                if b >= 2:
                    process(b - 2)
            process(nb - 2)
            process(nb - 1)

            @pl.when(mydeg)
            def _():
                pltpu.make_async_copy(
                    xh_hbm.at[0, pl.ds(0, cw), pl.ds(0, _DD)],
                    accd.at[pl.ds(0, cw)], dsem).wait()

        def body_common(c0, first):
            def stage_b():
                @pl.when(c0 + 1 < nch)
                def _():
                    stage(c0 + 1, src_b, dst_b, isb)

            def stage_a():
                @pl.when(c0 + 2 < nch)
                def _():
                    stage(c0 + 2, src_a, dst_a, isa)

            wait_stage(src_a, dst_a, isa)
            process_chunk(src_a, dst_a, (c0 % 2) == cid, first,
                          None if first else stage_b)
            wait_stage(src_b, dst_b, isb)
            process_chunk(src_b, dst_b, ((c0 + 1) % 2) == cid, False,
                          stage_a)

        body_common(0, True)

        def body(cc, carry):
            body_common(2 * cc, False)
            return carry
        lax.fori_loop(1, nch // 2, body, 0)

        for j in range(_NBUF):
            sz = bsizes[nb - _NBUF + j]
            pltpu.make_async_copy(xh_hbm.at[0, pl.ds(0, sz)],
                                  rows[j].at[pl.ds(0, sz)], ss[j]).wait()
        plsc.subcore_barrier()

        pltpu.sync_copy(
            accx.at[pl.ds(base, nr)],
            outx_hbm.at[pl.ds(base, nr), pl.ds(col0, fh)])
        pltpu.sync_copy(accd.at[pl.ds(base, nr)],
                        outd_hbm.at[cid, pl.ds(base, nr)])

    return sc_aggr


def _mlp(v, w1, b1, w2, b2):
    h = jnp.maximum(
        jnp.dot(v, w1[...], preferred_element_type=jnp.float32)
        + b1[...][None, :], 0.0)
    return (jnp.dot(h, w2[...], preferred_element_type=jnp.float32)
            + b2[...][None, :])


def _k_dense(n, f, *refs):
    (x_ref, ag_ref, d0_ref, d1_ref,
     c1w1, c1b1, c1w2, c1b2, c2w1, c2b1, c2w2, c2b2,
     c3w1, c3b1, c3w2, c3b2, mw1, mb1, mw2, mb2,
     u1w1, u1b1, u1w2, u1b2, u2w1, u2b1, u2w2, u2b2,
     u3w1, u3b1, u3w2, u3b2, out_ref) = refs
    y1 = x_ref[...] + ag_ref[...]
    deg = d0_ref[0, :, 0:1] + d1_ref[0, :, 0:1]
    x1 = _mlp(y1, c1w1, c1b1, c1w2, c1b2)
    xp1 = jnp.sum(x1, axis=0, keepdims=True) * (1.0 / n)
    m = 1.0 + deg[0:1, :]
    x2 = _mlp(m * xp1, c2w1, c2b1, c2w2, c2b2)
    x3 = _mlp(m * x2, c3w1, c3b1, c3w2, c3b2)
    xm = _mlp(m * x3, mw1, mb1, mw2, mb2)
    u1 = _mlp(m * jnp.concatenate([xm, x3], axis=1), u1w1, u1b1, u1w2, u1b2)
    u2 = _mlp(m * jnp.concatenate([u1, x2], axis=1), u2w1, u2b1, u2w2, u2b2)
    t1 = jnp.dot(u2, u3w1[0:64, :], preferred_element_type=jnp.float32)
    t0 = t1 + u3b1[...][None, :]
    z = jnp.dot(y1, u3w1[64:, :], preferred_element_type=jnp.float32)
    h = jnp.maximum(t0 + deg * t1 + z, 0.0)
    out_ref[...] = (jnp.dot(h, u3w2[...], preferred_element_type=jnp.float32)
                    + u3b2[...][None, :])


def kernel(x, edge_index, params):
    n, f = x.shape
    e = edge_index.shape[1]
    aggr, pd, _ = _make_sc_aggregate(n, f, e)(x, edge_index)

    flat_w = []
    for name in ('c1', 'c2', 'c3', 'mid', 'u1', 'u2', 'u3'):
        flat_w += list(params[name])
    ins = [x, aggr, pd, pd] + flat_w
    specs = []
    for i, a in enumerate(ins):
        if i in (2, 3):
            idx = 0 if i == 2 else 1
            specs.append(pl.BlockSpec(
                (1,) + a.shape[1:], functools.partial(
                    (lambda c, j: (c, 0, 0)), idx)))
        else:
            specs.append(pl.BlockSpec(
                a.shape, functools.partial(
                    (lambda nd, j: (0,) * nd), a.ndim)))
    out = pl.pallas_call(
        functools.partial(_k_dense, n, f),
        grid=(1,),
        in_specs=specs,
        out_specs=pl.BlockSpec((n, 64), lambda j: (0, 0)),
        out_shape=jax.ShapeDtypeStruct((n, 64), jnp.float32),
        compiler_params=pltpu.CompilerParams(
            vmem_limit_bytes=100 * 1024 * 1024),
    )(*ins)
    return out
